# Initial kernel scaffold; baseline (speedup 1.0000x reference)
import jax
import jax.numpy as jnp
from jax.experimental import pallas as pl


def kernel(h, us, ue, u, W_non, b_non, W_noise, b_noise, W_E, b_E, W_r, b_r):
    raise NotImplementedError("write your pallas kernel here")


# speedup vs baseline: 2.6003x; 2.6003x over previous
import functools

import jax
import jax.numpy as jnp
from jax.experimental import pallas as pl

DIM = 768
NE = 8
K2 = 2 * DIM

BT = 512
BD = 128
BDA = 256


def _beff_kernel(hcat_ref, wr_ref, br_ref,
                 wl_n_ref, bp_n_ref, wl_w_ref, bp_w_ref, wl_e_ref, bp_e_ref,
                 on_ref, ow_ref, oe_ref):
    r = jnp.dot(hcat_ref[...], wr_ref[...],
                preferred_element_type=jnp.float32) + br_ref[...]
    for wl_ref, bp_ref, o_ref in ((wl_n_ref, bp_n_ref, on_ref),
                                  (wl_w_ref, bp_w_ref, ow_ref),
                                  (wl_e_ref, bp_e_ref, oe_ref)):
        for e in range(NE):
            o_ref[e:e + 1, :] = (
                jnp.dot(r, wl_ref[e], preferred_element_type=jnp.float32)
                + bp_ref[e:e + 1, :])


def _moe_kernel(u_ref, wn_ref, ww_ref, we_ref, bn_ref, bw_ref, be_ref,
                nz_ref, o_ref):
    ub = u_ref[...]

    hh = []
    for e in range(NE):
        a = jnp.dot(ub, wn_ref[e], preferred_element_type=jnp.float32)
        w = jnp.dot(ub, ww_ref[e], preferred_element_type=jnp.float32)
        hh.append((a + bn_ref[e, :]) + (w + bw_ref[e, :]) * nz_ref[e])

    m1 = hh[0]
    for e in range(1, NE):
        m1 = jnp.maximum(m1, hh[e])
    sel1 = []
    found = jnp.zeros(m1.shape, dtype=jnp.bool_)
    for e in range(NE):
        s = (hh[e] == m1) & ~found
        found = found | s
        sel1.append(s)
    neg_inf = jnp.float32(-jnp.inf)
    x2 = [jnp.where(sel1[e], neg_inf, hh[e]) for e in range(NE)]
    m2 = x2[0]
    for e in range(1, NE):
        m2 = jnp.maximum(m2, x2[e])
    mask = []
    found = jnp.zeros(m1.shape, dtype=jnp.bool_)
    for e in range(NE):
        s = (x2[e] == m2) & ~found
        found = found | s
        mask.append(sel1[e] | s)

    logits = [jnp.where(mask[e], hh[e], 0.0) for e in range(NE)]
    logits = [jnp.where(l == 0.0, jnp.float32(-100000.0), l) for l in logits]
    mx = logits[0]
    for e in range(1, NE):
        mx = jnp.maximum(mx, logits[e])
    ex = [jnp.exp(logits[e] - mx) for e in range(NE)]
    ssum = ex[0]
    for e in range(1, NE):
        ssum = ssum + ex[e]

    acc = None
    for e in range(NE):
        ee = jnp.dot(ub, we_ref[e], preferred_element_type=jnp.float32)
        t = ex[e] * (ee + be_ref[e, :])
        acc = t if acc is None else acc + t
    o_ref[...] = acc / (ssum * jnp.float32(NE))


def _split_perm(w):
    wt = w.reshape(3 * DIM, DIM, NE).transpose(2, 0, 1)
    return wt[:, :K2], wt[:, K2:]


@jax.jit
def _run(h, us, ue, u, W_non, b_non, W_noise, b_noise, W_E, b_E, W_r, b_r):
    L = u.shape[1]
    f32 = jnp.float32

    hcat = jnp.concatenate([h[0], us[0], ue[0]], axis=-1)
    u2 = u[0]

    wu_n, wl_n = _split_perm(W_non)
    wu_w, wl_w = _split_perm(W_noise)
    wu_e, wl_e = _split_perm(W_E)
    bp_n = b_non.reshape(DIM, NE).T
    bp_w = b_noise.reshape(DIM, NE).T
    bp_e = b_E.reshape(DIM, NE).T

    noise = jax.random.normal(jax.random.key(42), (1, L, DIM, NE), dtype=f32)
    nz = noise[0].transpose(2, 0, 1)

    nda = DIM // BDA
    beff_spec = pl.BlockSpec((NE, BDA), lambda i: (0, i))
    beff_n, beff_w, beff_e = pl.pallas_call(
        _beff_kernel,
        grid=(nda,),
        in_specs=[
            pl.BlockSpec((1, 5 * DIM), lambda i: (0, 0)),
            pl.BlockSpec((5 * DIM, DIM), lambda i: (0, 0)),
            pl.BlockSpec((1, DIM), lambda i: (0, 0)),
            pl.BlockSpec((NE, DIM, BDA), lambda i: (0, 0, i)),
            beff_spec,
            pl.BlockSpec((NE, DIM, BDA), lambda i: (0, 0, i)),
            beff_spec,
            pl.BlockSpec((NE, DIM, BDA), lambda i: (0, 0, i)),
            beff_spec,
        ],
        out_specs=(beff_spec, beff_spec, beff_spec),
        out_shape=(jax.ShapeDtypeStruct((NE, DIM), f32),) * 3,
    )(hcat, W_r, b_r.reshape(1, DIM), wl_n, bp_n, wl_w, bp_w, wl_e, bp_e)

    nd, nt = DIM // BD, L // BT
    w_spec = pl.BlockSpec((NE, K2, BD), lambda i, j: (0, 0, i))
    b_spec = pl.BlockSpec((NE, BD), lambda i, j: (0, i))
    out = pl.pallas_call(
        _moe_kernel,
        grid=(nd, nt),
        in_specs=[
            pl.BlockSpec((BT, K2), lambda i, j: (j, 0)),
            w_spec, w_spec, w_spec,
            b_spec, b_spec, b_spec,
            pl.BlockSpec((NE, BT, BD), lambda i, j: (0, j, i)),
        ],
        out_specs=pl.BlockSpec((BT, BD), lambda i, j: (j, i)),
        out_shape=jax.ShapeDtypeStruct((L, DIM), f32),
    )(u2, wu_n, wu_w, wu_e, beff_n, beff_w, beff_e, nz)

    return out.reshape(1, L, DIM)


def kernel(h, us, ue, u, W_non, b_non, W_noise, b_noise, W_E, b_E, W_r, b_r):
    return _run(h, us, ue, u, W_non, b_non, W_noise, b_noise, W_E, b_E,
                W_r, b_r)


# speedup vs baseline: 3.6322x; 1.3968x over previous
import jax
import jax.numpy as jnp
from jax.experimental import pallas as pl

DIM = 768
NE = 8
L = 2048

BT = 256
BD = 128
BDA = 512

_DN0 = (((0,), (0,)), ((), ()))


def _dgt(a, b):
    return jax.lax.dot_general(a, b, _DN0, preferred_element_type=jnp.float32)


def _beff_kernel(hcat_ref, wr_ref, br_ref, wl_n_ref, b_n_ref, wl_w_ref,
                 b_w_ref, wl_e_ref, b_e_ref, on_ref, ow_ref, oe_ref):
    r = jnp.dot(hcat_ref[...], wr_ref[...],
                preferred_element_type=jnp.float32) + br_ref[...]
    for wl_ref, b_ref, o_ref in ((wl_n_ref, b_n_ref, on_ref),
                                 (wl_w_ref, b_w_ref, ow_ref),
                                 (wl_e_ref, b_e_ref, oe_ref)):
        o_ref[...] = jnp.dot(r, wl_ref[...],
                             preferred_element_type=jnp.float32) + b_ref[...]


def _moe_kernel(ut0_ref, ut1_ref, wn0_ref, wn1_ref, ww0_ref, ww1_ref,
                we0_ref, we1_ref, bn_ref, bw_ref, be_ref, nz_ref, o_ref):
    ut0 = ut0_ref[...]
    ut1 = ut1_ref[...]

    def proj(w0_ref, w1_ref, b_ref):
        m = _dgt(w0_ref[...], ut0) + _dgt(w1_ref[...], ut1)
        return m.reshape(BD, NE, BT) + b_ref[...][:, :, None]

    hh = proj(wn0_ref, wn1_ref, bn_ref) + proj(ww0_ref, ww1_ref,
                                               bw_ref) * nz_ref[...]

    idx = jax.lax.broadcasted_iota(jnp.int32, (BD, NE, BT), 1)
    m1 = jnp.max(hh, axis=1, keepdims=True)
    eq1 = hh == m1
    first1 = jnp.min(jnp.where(eq1, idx, NE), axis=1, keepdims=True)
    s1 = idx == first1
    x2 = jnp.where(s1, -jnp.inf, hh)
    m2 = jnp.max(x2, axis=1, keepdims=True)
    eq2 = x2 == m2
    first2 = jnp.min(jnp.where(eq2, idx, NE), axis=1, keepdims=True)
    mask = s1 | (idx == first2)

    z = jnp.where(mask, hh, 0.0)
    logits = jnp.where(z == 0.0, jnp.float32(-100000.0), z)
    mx = jnp.max(logits, axis=1, keepdims=True)
    ex = jnp.exp(logits - mx)
    ssum = jnp.sum(ex, axis=1)

    ew = proj(we0_ref, we1_ref, be_ref)
    num = jnp.sum(ex * ew, axis=1)
    o_ref[...] = num / (ssum * jnp.float32(NE))


@jax.jit
def _run(h, us, ue, u, W_non, b_non, W_noise, b_noise, W_E, b_E, W_r, b_r):
    f32 = jnp.float32

    hcat = jnp.concatenate([h[0], us[0], ue[0]], axis=-1)
    ut = u[0].T

    noise = jax.random.normal(jax.random.key(42), (1, L, DIM, NE), dtype=f32)
    nzt = noise[0].transpose(1, 2, 0)

    nda = NE * DIM // BDA
    row_spec = pl.BlockSpec((1, BDA), lambda i: (0, i))
    wlow_spec = pl.BlockSpec((DIM, BDA), lambda i: (2, i))
    beff_n, beff_w, beff_e = pl.pallas_call(
        _beff_kernel,
        grid=(nda,),
        in_specs=[
            pl.BlockSpec((1, 5 * DIM), lambda i: (0, 0)),
            pl.BlockSpec((5 * DIM, DIM), lambda i: (0, 0)),
            pl.BlockSpec((1, DIM), lambda i: (0, 0)),
            wlow_spec, row_spec, wlow_spec, row_spec, wlow_spec, row_spec,
        ],
        out_specs=(row_spec,) * 3,
        out_shape=(jax.ShapeDtypeStruct((1, NE * DIM), f32),) * 3,
    )(hcat, W_r, b_r.reshape(1, DIM), W_non, b_non.reshape(1, NE * DIM),
      W_noise, b_noise.reshape(1, NE * DIM), W_E, b_E.reshape(1, NE * DIM))

    nd, nt = DIM // BD, L // BT
    ut_spec0 = pl.BlockSpec((DIM, BT), lambda i, j: (0, j))
    ut_spec1 = pl.BlockSpec((DIM, BT), lambda i, j: (1, j))
    w_spec0 = pl.BlockSpec((DIM, NE * BD), lambda i, j: (0, i))
    w_spec1 = pl.BlockSpec((DIM, NE * BD), lambda i, j: (1, i))
    b_spec = pl.BlockSpec((BD, NE), lambda i, j: (i, 0))
    out_t = pl.pallas_call(
        _moe_kernel,
        grid=(nd, nt),
        in_specs=[
            ut_spec0, ut_spec1,
            w_spec0, w_spec1, w_spec0, w_spec1, w_spec0, w_spec1,
            b_spec, b_spec, b_spec,
            pl.BlockSpec((BD, NE, BT), lambda i, j: (i, 0, j)),
        ],
        out_specs=pl.BlockSpec((BD, BT), lambda i, j: (i, j)),
        out_shape=jax.ShapeDtypeStruct((DIM, L), f32),
    )(ut, ut, W_non, W_non, W_noise, W_noise, W_E, W_E,
      beff_n.reshape(DIM, NE), beff_w.reshape(DIM, NE),
      beff_e.reshape(DIM, NE), nzt)

    return out_t.T.reshape(1, L, DIM)


def kernel(h, us, ue, u, W_non, b_non, W_noise, b_noise, W_E, b_E, W_r, b_r):
    return _run(h, us, ue, u, W_non, b_non, W_noise, b_noise, W_E, b_E,
                W_r, b_r)
